# baseline (speedup 1.0000x reference)
import jax
import jax.numpy as jnp
from jax.experimental import pallas as pl
from jax.experimental.pallas import tpu as pltpu


_ROWS, _COLS = 16384, 128
_CHUNKS = (512, 2048, 4608, 4608, 3072, 1536)
assert sum(_CHUNKS) == _ROWS
_OFFS = tuple(sum(_CHUNKS[:i]) for i in range(len(_CHUNKS)))
_N = len(_CHUNKS)


def _copy_kernel(x_hbm, o_hbm, buf, sem_in, sem_out):
    ins = []
    for i in range(_N):
        cp = pltpu.make_async_copy(
            x_hbm.at[pl.ds(_OFFS[i], _CHUNKS[i]), :],
            buf.at[pl.ds(_OFFS[i], _CHUNKS[i]), :],
            sem_in.at[i],
        )
        cp.start()
        ins.append(cp)

    outs = []
    for i in range(_N):
        ins[i].wait()
        if i == 0:
            lane = jax.lax.iota(jnp.int32, _COLS)
            head = buf[0, :]
            buf[0, :] = jnp.where(lane == 0, jnp.float32(3.0), head)
        cp = pltpu.make_async_copy(
            buf.at[pl.ds(_OFFS[i], _CHUNKS[i]), :],
            o_hbm.at[pl.ds(_OFFS[i], _CHUNKS[i]), :],
            sem_out.at[i],
        )
        cp.start()
        outs.append(cp)

    for cp in outs:
        cp.wait()


def kernel(x):
    return pl.pallas_call(
        _copy_kernel,
        in_specs=[pl.BlockSpec(memory_space=pl.ANY)],
        out_specs=pl.BlockSpec(memory_space=pl.ANY),
        out_shape=jax.ShapeDtypeStruct((_ROWS, _COLS), x.dtype),
        scratch_shapes=[
            pltpu.VMEM((_ROWS, _COLS), jnp.float32),
            pltpu.SemaphoreType.DMA((_N,)),
            pltpu.SemaphoreType.DMA((_N,)),
        ],
    )(x)


# speedup vs baseline: 1.0207x; 1.0207x over previous
import jax
import jax.numpy as jnp
from jax.experimental import pallas as pl
from jax.experimental.pallas import tpu as pltpu


_ROWS, _COLS = 16384, 128
_CHUNKS = (384, 1152, 3328, 4352, 3840, 2048, 1024, 256)
assert sum(_CHUNKS) == _ROWS
_OFFS = tuple(sum(_CHUNKS[:i]) for i in range(len(_CHUNKS)))
_N = len(_CHUNKS)


def _copy_kernel(x_hbm, o_hbm, buf, sem_in, sem_out):
    ins = []
    for i in range(_N):
        cp = pltpu.make_async_copy(
            x_hbm.at[pl.ds(_OFFS[i], _CHUNKS[i]), :],
            buf.at[pl.ds(_OFFS[i], _CHUNKS[i]), :],
            sem_in.at[i],
        )
        cp.start()
        ins.append(cp)

    outs = []
    for i in range(_N):
        ins[i].wait()
        if i == 0:
            lane = jax.lax.iota(jnp.int32, _COLS)
            head = buf[0, :]
            buf[0, :] = jnp.where(lane == 0, jnp.float32(3.0), head)
        cp = pltpu.make_async_copy(
            buf.at[pl.ds(_OFFS[i], _CHUNKS[i]), :],
            o_hbm.at[pl.ds(_OFFS[i], _CHUNKS[i]), :],
            sem_out.at[i],
        )
        cp.start()
        outs.append(cp)

    for cp in outs:
        cp.wait()


def kernel(x):
    return pl.pallas_call(
        _copy_kernel,
        in_specs=[pl.BlockSpec(memory_space=pl.ANY)],
        out_specs=pl.BlockSpec(memory_space=pl.ANY),
        out_shape=jax.ShapeDtypeStruct((_ROWS, _COLS), x.dtype),
        scratch_shapes=[
            pltpu.VMEM((_ROWS, _COLS), jnp.float32),
            pltpu.SemaphoreType.DMA((_N,)),
            pltpu.SemaphoreType.DMA((_N,)),
        ],
    )(x)
